# Initial kernel scaffold; baseline (speedup 1.0000x reference)
import jax
import jax.numpy as jnp
from jax.experimental import pallas as pl


def kernel(x, edge_index, batch, W_l, W_r, b):
    raise NotImplementedError("write your pallas kernel here")


# speedup vs baseline: 4.8196x; 4.8196x over previous
import functools

import jax
import jax.numpy as jnp
from jax import lax
from jax.experimental import pallas as pl
from jax.experimental.pallas import tpu as pltpu
from jax.experimental.pallas import tpu_sc as plsc

_LANES = 128


def _make_sc_aggregate(NP, D, EP_W, NB, NC, NS):
  rows_per_tile = NP // NS
  n_zero_blocks = rows_per_tile // _LANES
  mesh = plsc.VectorSubcoreMesh(core_axis_name="c", subcore_axis_name="s")

  @functools.partial(
      pl.kernel,
      out_type=(
          jax.ShapeDtypeStruct((NP, D), jnp.float32),
          jax.ShapeDtypeStruct((NP, D), jnp.float32),
          jax.ShapeDtypeStruct((NP,), jnp.float32),
          jax.ShapeDtypeStruct((NP,), jnp.float32),
      ),
      mesh=mesh,
      scratch_types=[
          pltpu.VMEM((_LANES,), jnp.int32),
          pltpu.VMEM((_LANES,), jnp.int32),
          pltpu.VMEM((_LANES, D), jnp.float32),
          pltpu.VMEM((_LANES, D), jnp.float32),
          pltpu.VMEM((rows_per_tile,), jnp.float32),
          pltpu.VMEM((_LANES,), jnp.float32),
          pltpu.VMEM_SHARED((NP, D), jnp.float32),
          pltpu.VMEM_SHARED((NP,), jnp.float32),
          pltpu.SemaphoreType.DMA,
      ],
  )
  def sc_kernel(src_hbm, dst_hbm, x_hbm, z2_hbm, z1_hbm, ones_hbm,
                acc0_hbm, acc1_hbm, deg0_hbm, deg1_hbm,
                src_v, dst_v, buf, z2, z1, ones_v, acc_s, deg_s, sem):
    cid = lax.axis_index("c")
    sid = lax.axis_index("s")
    wid = sid * NC + cid
    row0 = sid * rows_per_tile

    pltpu.sync_copy(z2_hbm, z2)
    pltpu.sync_copy(z1_hbm, z1)
    pltpu.sync_copy(ones_hbm, ones_v)
    for r in range(n_zero_blocks):
      pltpu.sync_copy(z2, acc_s.at[pl.ds(row0 + r * _LANES, _LANES)])
    pltpu.sync_copy(z1, deg_s.at[pl.ds(row0, rows_per_tile)])
    plsc.subcore_barrier()

    base = wid * EP_W

    def body(j, carry):
      off = base + j * _LANES
      pltpu.sync_copy(src_hbm.at[pl.ds(off, _LANES)], src_v)
      pltpu.sync_copy(dst_hbm.at[pl.ds(off, _LANES)], dst_v)
      pltpu.async_copy(x_hbm.at[src_v], buf, sem).wait()
      pltpu.sync_copy(buf, acc_s.at[dst_v], add=True)
      pltpu.sync_copy(ones_v, deg_s.at[dst_v], add=True)
      return carry

    lax.fori_loop(0, NB, body, 0)
    plsc.subcore_barrier()

    @pl.when(cid == 0)
    def _():
      pltpu.sync_copy(acc_s.at[pl.ds(row0, rows_per_tile)],
                      acc0_hbm.at[pl.ds(row0, rows_per_tile)])
      pltpu.sync_copy(deg_s.at[pl.ds(row0, rows_per_tile)],
                      deg0_hbm.at[pl.ds(row0, rows_per_tile)])

    @pl.when(cid == 1)
    def _():
      pltpu.sync_copy(acc_s.at[pl.ds(row0, rows_per_tile)],
                      acc1_hbm.at[pl.ds(row0, rows_per_tile)])
      pltpu.sync_copy(deg_s.at[pl.ds(row0, rows_per_tile)],
                      deg1_hbm.at[pl.ds(row0, rows_per_tile)])

  return sc_kernel


def _tc_finish(acc0_ref, acc1_ref, deg0_ref, deg1_ref, x_ref, wl_ref, wr_ref,
               b_ref, out_ref):
  deg = jnp.maximum(deg0_ref[...] + deg1_ref[...], 1.0)
  agg = (acc0_ref[...] + acc1_ref[...]) / deg
  out = (jnp.dot(agg, wl_ref[...], preferred_element_type=jnp.float32)
         + jnp.dot(x_ref[...], wr_ref[...], preferred_element_type=jnp.float32)
         + b_ref[...])
  norm = jnp.sqrt(jnp.sum(out * out, axis=1, keepdims=True))
  out = out / jnp.maximum(norm, 1e-12)
  out_ref[...] = jnp.maximum(out, 0.0)


def kernel(x, edge_index, batch, W_l, W_r, b):
  del batch
  N, D = x.shape
  E = edge_index.shape[1]
  NC, NS = 2, 16
  NW = NC * NS

  NP = ((N + 1 + NS * _LANES - 1) // (NS * _LANES)) * (NS * _LANES)
  E_pad = ((E + NW * _LANES - 1) // (NW * _LANES)) * (NW * _LANES)
  EP_W = E_pad // NW
  NB = EP_W // _LANES

  src = jnp.concatenate(
      [edge_index[0], jnp.zeros((E_pad - E,), jnp.int32)])
  dst = jnp.concatenate(
      [edge_index[1], jnp.full((E_pad - E,), N, jnp.int32)])
  x_pad = jnp.pad(x, ((0, NP - N), (0, 0)))
  z2 = jnp.zeros((_LANES, D), jnp.float32)
  z1 = jnp.zeros((NP // NS,), jnp.float32)
  ones = jnp.ones((_LANES,), jnp.float32)

  sc = _make_sc_aggregate(NP, D, EP_W, NB, NC, NS)
  acc0, acc1, deg0, deg1 = sc(src, dst, x_pad, z2, z1, ones)

  R = 512
  grid = (NP // R,)
  out = pl.pallas_call(
      _tc_finish,
      grid=grid,
      in_specs=[
          pl.BlockSpec((R, D), lambda i: (i, 0)),
          pl.BlockSpec((R, D), lambda i: (i, 0)),
          pl.BlockSpec((R, 1), lambda i: (i, 0)),
          pl.BlockSpec((R, 1), lambda i: (i, 0)),
          pl.BlockSpec((R, D), lambda i: (i, 0)),
          pl.BlockSpec((D, D), lambda i: (0, 0)),
          pl.BlockSpec((D, D), lambda i: (0, 0)),
          pl.BlockSpec((1, D), lambda i: (0, 0)),
      ],
      out_specs=pl.BlockSpec((R, D), lambda i: (i, 0)),
      out_shape=jax.ShapeDtypeStruct((NP, D), jnp.float32),
  )(acc0, acc1, deg0.reshape(NP, 1), deg1.reshape(NP, 1), x_pad, W_l, W_r,
    b.reshape(1, D))
  return out[:N]
